# baseline (speedup 1.0000x reference)
import functools

import jax
import jax.numpy as jnp
from jax import lax
from jax.experimental import pallas as pl
from jax.experimental.pallas import tpu as pltpu
from jax.experimental.pallas import tpu_sc as plsc


@functools.lru_cache(maxsize=None)
def _make_transpose(VP, D):
    mesh = plsc.VectorSubcoreMesh(core_axis_name="c", subcore_axis_name="s")
    info = plsc.get_sparse_core_info()
    NC = info.num_cores
    NW = NC * info.num_subcores
    D2 = 2 * D
    NT = VP // D2
    assert NT * D2 == VP and D % 16 == 0
    BASE = NT // NW
    EXTRA = NT - BASE * NW
    assert BASE % 2 == 0
    NPAIRS = BASE // 2

    @functools.partial(
        pl.kernel,
        mesh=mesh,
        out_type=jax.ShapeDtypeStruct((VP, D2), jnp.float32),
        compiler_params=pltpu.CompilerParams(
            use_tc_tiling_on_sc=False, needs_layout_passes=False
        ),
        scratch_types=[
            pltpu.VMEM((2, D // 8, 8 * D2), jnp.float32),
            pltpu.VMEM((2, D2, D), jnp.float32),
            pltpu.SemaphoreType.DMA,
            pltpu.SemaphoreType.DMA,
            pltpu.SemaphoreType.DMA,
            pltpu.SemaphoreType.DMA,
        ],
    )
    def tk(emb4, out, bufA, bufB, ls0, ls1, ss0, ss1):
        w = lax.axis_index("s") * NC + lax.axis_index("c")
        start = BASE * w + jnp.minimum(w, EXTRA)
        lsem = (ls0, ls1)
        ssem = (ss0, ss1)
        iota = lax.iota(jnp.int32, 16)

        def load_start(s, blk):
            blk = jnp.minimum(blk, NT - 1)
            pltpu.async_copy(emb4.at[:, blk, :], bufA.at[s], lsem[s])

        def load_wait(s):
            pltpu.make_async_copy(
                emb4.at[:, 0, :], bufA.at[s], lsem[s]
            ).wait()

        def store_start(s, blk):
            pltpu.async_copy(
                bufB.at[s], out.at[pl.ds(blk * D2, D2), pl.ds(0, D)], ssem[s]
            )

        def store_wait(s):
            pltpu.make_async_copy(
                bufB.at[s], out.at[pl.ds(0, D2), pl.ds(0, D)], ssem[s]
            ).wait()


        def transpose(s):
            for d in range(D):
                dt, ds = divmod(d, 8)
                cv = jnp.full((16,), d, jnp.int32)
                for tg in range(D2 // 16):
                    v = bufA[s, dt, pl.ds(ds * D2 + 16 * tg, 16)]
                    plsc.store_scatter(bufB.at[s], [iota + 16 * tg, cv], v)

        load_start(0, start)
        load_start(1, start + 1)

        def pbody(p, c):
            blk0 = start + 2 * p
            for s in range(2):
                load_wait(s)

                @pl.when(p > 0)
                def _():
                    store_wait(s)

                transpose(s)
                load_start(s, blk0 + s + 2)
                store_start(s, blk0 + s)
            return c

        lax.fori_loop(0, NPAIRS, pbody, 0)
        store_wait(0)
        store_wait(1)
        load_wait(0)
        load_wait(1)

        @pl.when(w < EXTRA)
        def _():
            blk = start + BASE
            pltpu.sync_copy(emb4.at[:, blk, :], bufA.at[0])
            transpose(0)
            pltpu.sync_copy(bufB.at[0], out.at[pl.ds(blk * D2, D2), pl.ds(0, D)])

    return tk


@functools.lru_cache(maxsize=None)
def _make_gather(V2, D, NW, n_ch, CH, NBUF):
    mesh = plsc.VectorSubcoreMesh(core_axis_name="c", subcore_axis_name="s")
    info = plsc.get_sparse_core_info()
    NC = info.num_cores
    n_grp = n_ch // NBUF
    assert n_grp * NBUF == n_ch and n_grp % 2 == 0 and n_grp >= 6
    n_pairs = n_grp // 2
    n_chunks = NW * n_ch

    @functools.partial(
        pl.kernel,
        mesh=mesh,
        out_type=jax.ShapeDtypeStruct((n_chunks, CH, 2 * D), jnp.float32),
        compiler_params=pltpu.CompilerParams(use_tc_tiling_on_sc=False),
        scratch_types=[
            pltpu.VMEM((n_ch, CH), jnp.int32),
            pltpu.VMEM((2, NBUF, CH, D), jnp.float32),
            pltpu.SemaphoreType.DMA,
            pltpu.SemaphoreType.DMA,
            pltpu.SemaphoreType.DMA,
            pltpu.SemaphoreType.DMA,
        ],
    )
    def k(ids_hbm, table_hbm, out_hbm, idx_v, rows, gsA, gsB, ssA, ssB):
        cid = lax.axis_index("c")
        sid = lax.axis_index("s")
        wid = sid * NC + cid
        base = wid * n_ch
        pltpu.sync_copy(ids_hbm.at[wid], idx_v)
        gsem = (gsA, gsB)
        ssem = (ssA, ssB)

        def g_start(s, b, j):
            pltpu.async_copy(table_hbm.at[idx_v.at[j]], rows.at[s, b], gsem[s])

        def g_wait(s, b):
            pltpu.make_async_copy(
                table_hbm.at[pl.ds(0, CH)], rows.at[s, b], gsem[s]
            ).wait()

        def s_start(s, b, j):
            pltpu.async_copy(
                rows.at[s, b], out_hbm.at[base + j, :, pl.ds(0, D)], ssem[s]
            )

        def s_wait(s, b):
            pltpu.make_async_copy(
                rows.at[s, b], out_hbm.at[base, :, pl.ds(0, D)], ssem[s]
            ).wait()

        for b in range(NBUF):
            g_start(0, b, b)

        for b in range(NBUF):
            g_wait(0, b)
        for b in range(NBUF):
            g_start(1, b, NBUF + b)
        for b in range(NBUF):
            s_start(0, b, b)
        for b in range(NBUF):
            g_wait(1, b)
        for b in range(NBUF):
            s_wait(0, b)
        for b in range(NBUF):
            g_start(0, b, 2 * NBUF + b)
        for b in range(NBUF):
            s_start(1, b, NBUF + b)

        def pair_body(p, carry):
            g0 = 2 * p
            for b in range(NBUF):
                g_wait(0, b)
            for b in range(NBUF):
                s_wait(1, b)
            for b in range(NBUF):
                g_start(1, b, (g0 + 1) * NBUF + b)
            for b in range(NBUF):
                s_start(0, b, g0 * NBUF + b)
            for b in range(NBUF):
                g_wait(1, b)
            for b in range(NBUF):
                s_wait(0, b)
            for b in range(NBUF):
                g_start(0, b, (g0 + 2) * NBUF + b)
            for b in range(NBUF):
                s_start(1, b, (g0 + 1) * NBUF + b)
            return carry

        lax.fori_loop(1, n_pairs - 1, pair_body, 0)

        g0 = n_grp - 2
        for b in range(NBUF):
            g_wait(0, b)
        for b in range(NBUF):
            s_wait(1, b)
        for b in range(NBUF):
            g_start(1, b, (g0 + 1) * NBUF + b)
        for b in range(NBUF):
            s_start(0, b, g0 * NBUF + b)
        for b in range(NBUF):
            g_wait(1, b)
        for b in range(NBUF):
            s_wait(0, b)
        for b in range(NBUF):
            s_start(1, b, (g0 + 1) * NBUF + b)
        for b in range(NBUF):
            s_wait(1, b)

    return k


def kernel(input_ids, input_mask, emb_weight):
    B, S = input_ids.shape
    V, D = emb_weight.shape
    N = B * S
    NW = 32
    CH = 128
    NBUF = 4
    n_ch = N // (NW * CH)
    assert N == NW * n_ch * CH
    D2 = 2 * D
    VP = ((V + D2 - 1) // D2) * D2
    embp = jnp.pad(emb_weight, ((0, VP - V), (0, 0)))
    emb4 = (
        embp.T.reshape(D // 8, 8, VP // D2, D2)
        .transpose(0, 2, 1, 3)
        .reshape(D // 8, VP // D2, 8 * D2)
    )
    t2 = _make_transpose(VP, D)(emb4).reshape(2 * VP, D)
    ids2 = (input_ids.reshape(N) * 2).reshape(NW, n_ch, CH)
    out3 = _make_gather(2 * V, D, NW, n_ch, CH, NBUF)(ids2, t2)
    out = out3.reshape(N, 2 * D)[:, :D].reshape(B, S, D)
    return (out, input_mask)


# speedup vs baseline: 5.2747x; 5.2747x over previous
import functools

import jax
import jax.numpy as jnp
from jax import lax
from jax.experimental import pallas as pl
from jax.experimental.pallas import tpu as pltpu
from jax.experimental.pallas import tpu_sc as plsc


@functools.lru_cache(maxsize=None)
def _make_gather(V2, D, NW, n_ch, CH, NBUF):
    mesh = plsc.VectorSubcoreMesh(core_axis_name="c", subcore_axis_name="s")
    info = plsc.get_sparse_core_info()
    NC = info.num_cores
    n_grp = n_ch // NBUF
    assert n_grp * NBUF == n_ch and n_grp % 2 == 0 and n_grp >= 6
    n_pairs = n_grp // 2
    n_chunks = NW * n_ch

    @functools.partial(
        pl.kernel,
        mesh=mesh,
        out_type=jax.ShapeDtypeStruct((n_chunks, CH, 2 * D), jnp.float32),
        compiler_params=pltpu.CompilerParams(use_tc_tiling_on_sc=False),
        scratch_types=[
            pltpu.VMEM((n_ch, CH), jnp.int32),
            pltpu.VMEM((2, NBUF, CH, D), jnp.float32),
            pltpu.SemaphoreType.DMA,
            pltpu.SemaphoreType.DMA,
            pltpu.SemaphoreType.DMA,
            pltpu.SemaphoreType.DMA,
        ],
    )
    def k(ids_hbm, table_hbm, out_hbm, idx_v, rows, gsA, gsB, ssA, ssB):
        cid = lax.axis_index("c")
        sid = lax.axis_index("s")
        wid = sid * NC + cid
        base = wid * n_ch
        pltpu.sync_copy(ids_hbm.at[wid], idx_v)
        gsem = (gsA, gsB)
        ssem = (ssA, ssB)

        def g_start(s, b, j):
            pltpu.async_copy(table_hbm.at[idx_v.at[j]], rows.at[s, b], gsem[s])

        def g_wait(s, b):
            pltpu.make_async_copy(
                table_hbm.at[pl.ds(0, CH)], rows.at[s, b], gsem[s]
            ).wait()

        def s_start(s, b, j):
            pltpu.async_copy(
                rows.at[s, b], out_hbm.at[base + j, :, pl.ds(0, D)], ssem[s]
            )

        def s_wait(s, b):
            pltpu.make_async_copy(
                rows.at[s, b], out_hbm.at[base, :, pl.ds(0, D)], ssem[s]
            ).wait()

        for b in range(NBUF):
            g_start(0, b, b)

        for b in range(NBUF):
            g_wait(0, b)
        for b in range(NBUF):
            g_start(1, b, NBUF + b)
        for b in range(NBUF):
            s_start(0, b, b)
        for b in range(NBUF):
            g_wait(1, b)
        for b in range(NBUF):
            s_wait(0, b)
        for b in range(NBUF):
            g_start(0, b, 2 * NBUF + b)
        for b in range(NBUF):
            s_start(1, b, NBUF + b)

        def pair_body(p, carry):
            g0 = 2 * p
            for b in range(NBUF):
                g_wait(0, b)
            for b in range(NBUF):
                s_wait(1, b)
            for b in range(NBUF):
                g_start(1, b, (g0 + 1) * NBUF + b)
            for b in range(NBUF):
                s_start(0, b, g0 * NBUF + b)
            for b in range(NBUF):
                g_wait(1, b)
            for b in range(NBUF):
                s_wait(0, b)
            for b in range(NBUF):
                g_start(0, b, (g0 + 2) * NBUF + b)
            for b in range(NBUF):
                s_start(1, b, (g0 + 1) * NBUF + b)
            return carry

        lax.fori_loop(1, n_pairs - 1, pair_body, 0)

        g0 = n_grp - 2
        for b in range(NBUF):
            g_wait(0, b)
        for b in range(NBUF):
            s_wait(1, b)
        for b in range(NBUF):
            g_start(1, b, (g0 + 1) * NBUF + b)
        for b in range(NBUF):
            s_start(0, b, g0 * NBUF + b)
        for b in range(NBUF):
            g_wait(1, b)
        for b in range(NBUF):
            s_wait(0, b)
        for b in range(NBUF):
            s_start(1, b, (g0 + 1) * NBUF + b)
        for b in range(NBUF):
            s_wait(1, b)

    return k


def kernel(input_ids, input_mask, emb_weight):
    B, S = input_ids.shape
    V, D = emb_weight.shape
    N = B * S
    NW = 32
    CH = 128
    NBUF = 4
    n_ch = N // (NW * CH)
    assert N == NW * n_ch * CH
    t2 = jnp.pad(emb_weight, ((0, 0), (0, D))).reshape(2 * V, D)
    ids2 = (input_ids.reshape(N) * 2).reshape(NW, n_ch, CH)
    out3 = _make_gather(2 * V, D, NW, n_ch, CH, NBUF)(ids2, t2)
    out = out3.reshape(N, 2 * D)[:, :D].reshape(B, S, D)
    return (out, input_mask)
